# baseline (speedup 1.0000x reference)
import jax
import jax.numpy as jnp
from jax import lax
from jax.experimental import pallas as pl
from jax.experimental.pallas import tpu as pltpu
from jax.experimental.pallas import tpu_sc as plsc

V = 50265
H = 768
B, P, T = 16, 8, 129
NSEQ = B + B * P
TOK = T - 1
NQ = 4
QT = TOK // NQ
HALF = NSEQ // 2
SPT = HALF // 8
BS = 8


def _sc_gather_body(ids_hbm, tab_hbm, out_hbm, ids_v, rows_v,
                    g0, g1, w0, w1):
    wid = lax.axis_index("c") * 16 + lax.axis_index("s")
    q = wid // 8
    seq_base = (wid % 8) * SPT

    pltpu.sync_copy(ids_hbm.at[wid], ids_v)
    gsem = (g0, g1)
    wsem = (w0, w1)

    pltpu.async_copy(tab_hbm.at[ids_v.at[0]], rows_v.at[0], g0)
    pltpu.async_copy(tab_hbm.at[ids_v.at[1]], rows_v.at[1], g1)

    for j in range(SPT):
        buf = j % 2
        dst = out_hbm.at[q, seq_base + j]
        pltpu.make_async_copy(
            tab_hbm.at[ids_v.at[j]], rows_v.at[buf], gsem[buf]).wait()
        pltpu.async_copy(rows_v.at[buf], dst, wsem[buf])
        if j + 2 < SPT:
            pltpu.make_async_copy(rows_v.at[buf], dst, wsem[buf]).wait()
            pltpu.async_copy(
                tab_hbm.at[ids_v.at[j + 2]], rows_v.at[buf], gsem[buf])

    for j in (SPT - 2, SPT - 1):
        buf = j % 2
        dst = out_hbm.at[q, seq_base + j]
        pltpu.make_async_copy(rows_v.at[buf], dst, wsem[buf]).wait()


def _make_sc_gather():
    mesh = plsc.VectorSubcoreMesh(core_axis_name="c", subcore_axis_name="s")
    return pl.kernel(
        _sc_gather_body,
        out_type=jax.ShapeDtypeStruct((NQ, HALF, QT, H), jnp.float32),
        mesh=mesh,
        scratch_types=[
            pltpu.VMEM((SPT, QT), jnp.int32),
            pltpu.VMEM((2, QT, H), jnp.float32),
            pltpu.SemaphoreType.DMA,
            pltpu.SemaphoreType.DMA,
            pltpu.SemaphoreType.DMA,
            pltpu.SemaphoreType.DMA,
        ],
    )


def _pool_body(g_ref, c_ref, gam_ref, bet_ref, out_ref):
    e = g_ref[...] + c_ref[...][:, None]
    mu = jnp.mean(e, axis=-1, keepdims=True)
    var = jnp.mean(e * e, axis=-1, keepdims=True) - mu * mu
    w = lax.rsqrt(var + jnp.float32(1e-5))
    s = jnp.sum(e * w, axis=(0, 2)) - jnp.sum(
        mu * w, axis=(0, 2))
    out_ref[...] = (s * jnp.float32(1.0 / TOK)) * gam_ref[...] + bet_ref[...]


def _tail_body(pa_ref, pb_ref, w_ref, wb_ref, out_ref):
    pa = pa_ref[...]
    pb = pb_ref[...]
    ph = pa[:B]
    pp = jnp.concatenate([pa[B:], pb], axis=0).reshape(B, P, H)
    hist = lax.dot_general(ph, w_ref[...], (((1,), (1,)), ((), ())),
                           precision=lax.Precision.HIGHEST,
                           preferred_element_type=jnp.float32)
    hist = hist + wb_ref[...]
    diff = pp - hist[:, None, :]
    d2 = jnp.sum(diff * diff, axis=-1)
    feats = -jnp.sqrt(d2)
    m = jnp.max(feats, axis=-1, keepdims=True)
    ex = jnp.exp(feats - m)
    out_ref[...] = ex / jnp.sum(ex, axis=-1, keepdims=True)


def _pool_call(gathered, c, ln_g, ln_b):
    return pl.pallas_call(
        _pool_body,
        grid=(HALF // BS,),
        in_specs=[
            pl.BlockSpec((NQ, BS, QT, H), lambda i: (0, i, 0, 0)),
            pl.BlockSpec((NQ, QT, H), lambda i: (0, 0, 0)),
            pl.BlockSpec((1, H), lambda i: (0, 0)),
            pl.BlockSpec((1, H), lambda i: (0, 0)),
        ],
        out_specs=pl.BlockSpec((BS, H), lambda i: (i, 0)),
        out_shape=jax.ShapeDtypeStruct((HALF, H), jnp.float32),
    )(gathered, c, ln_g, ln_b)


def kernel(persona, history, word_emb, pos_emb, tok_type_emb, ln_g, ln_b, W, b):
    ids = jnp.concatenate(
        [history[:, 1:].reshape(B, TOK),
         persona[:, :, 1:].reshape(B * P, TOK)], axis=0).astype(jnp.int32)
    ids = ids.reshape(2, HALF, NQ, QT).transpose(0, 2, 1, 3).reshape(
        2, 32, SPT, QT)
    c = (pos_emb[2:2 + TOK] + tok_type_emb[0]).reshape(NQ, QT, H)
    gam = ln_g.reshape(1, H)
    bet = ln_b.reshape(1, H)

    sc_gather = _make_sc_gather()
    ga = sc_gather(ids[0], word_emb)
    gb = sc_gather(ids[1], word_emb)
    pa = _pool_call(ga, c, gam, bet)
    pb = _pool_call(gb, c, gam, bet)

    return pl.pallas_call(
        _tail_body,
        out_shape=jax.ShapeDtypeStruct((B, P), jnp.float32),
    )(pa, pb, W, b)


# speedup vs baseline: 1.0059x; 1.0059x over previous
import jax
import jax.numpy as jnp
from jax import lax
from jax.experimental import pallas as pl
from jax.experimental.pallas import tpu as pltpu
from jax.experimental.pallas import tpu_sc as plsc

V = 50265
H = 768
B, P, T = 16, 8, 129
NSEQ = B + B * P
TOK = T - 1
NQ = 4
QT = TOK // NQ
SPT = NSEQ // 8
BS = 16
NBUF = 4


def _sc_gather_body(ids_hbm, tab_hbm, out_hbm, ids_v, rows_v,
                    g0, g1, g2, g3, w0, w1, w2, w3):
    wid = lax.axis_index("c") * 16 + lax.axis_index("s")
    q = wid // 8
    seq_base = (wid % 8) * SPT

    pltpu.sync_copy(ids_hbm.at[wid], ids_v)
    gsem = (g0, g1, g2, g3)
    wsem = (w0, w1, w2, w3)

    def wdst(j):
        return out_hbm.at[q, seq_base + j]

    pltpu.async_copy(tab_hbm.at[ids_v.at[0]], rows_v.at[0], g0)
    pltpu.async_copy(tab_hbm.at[ids_v.at[1]], rows_v.at[1], g1)

    for j in range(SPT):
        buf = j % NBUF
        pltpu.make_async_copy(
            tab_hbm.at[ids_v.at[j]], rows_v.at[buf], gsem[buf]).wait()
        pltpu.async_copy(rows_v.at[buf], wdst(j), wsem[buf])
        nxt = j + 2
        if nxt < SPT:
            nbuf = nxt % NBUF
            if nxt >= NBUF:
                pltpu.make_async_copy(
                    rows_v.at[nbuf], wdst(nxt - NBUF), wsem[nbuf]).wait()
            pltpu.async_copy(
                tab_hbm.at[ids_v.at[nxt]], rows_v.at[nbuf], gsem[nbuf])

    for j in range(SPT - NBUF, SPT):
        buf = j % NBUF
        pltpu.make_async_copy(rows_v.at[buf], wdst(j), wsem[buf]).wait()


def _make_sc_gather():
    mesh = plsc.VectorSubcoreMesh(core_axis_name="c", subcore_axis_name="s")
    return pl.kernel(
        _sc_gather_body,
        out_type=jax.ShapeDtypeStruct((NQ, NSEQ, QT, H), jnp.float32),
        mesh=mesh,
        scratch_types=[
            pltpu.VMEM((SPT, QT), jnp.int32),
            pltpu.VMEM((NBUF, QT, H), jnp.float32),
            pltpu.SemaphoreType.DMA,
            pltpu.SemaphoreType.DMA,
            pltpu.SemaphoreType.DMA,
            pltpu.SemaphoreType.DMA,
            pltpu.SemaphoreType.DMA,
            pltpu.SemaphoreType.DMA,
            pltpu.SemaphoreType.DMA,
            pltpu.SemaphoreType.DMA,
        ],
    )


def _pool_body(g_ref, c_ref, gam_ref, bet_ref, out_ref):
    e = g_ref[...] + c_ref[...][:, None]
    mu = jnp.mean(e, axis=-1, keepdims=True)
    var = jnp.mean(e * e, axis=-1, keepdims=True) - mu * mu
    w = lax.rsqrt(var + jnp.float32(1e-5))
    s = jnp.sum(e * w, axis=(0, 2)) - jnp.sum(
        mu * w, axis=(0, 2))
    out_ref[...] = (s * jnp.float32(1.0 / TOK)) * gam_ref[...] + bet_ref[...]


def _tail_body(pooled_ref, w_ref, wb_ref, out_ref):
    pooled = pooled_ref[...]
    ph = pooled[:B]
    pp = pooled[B:].reshape(B, P, H)
    hist = lax.dot_general(ph, w_ref[...], (((1,), (1,)), ((), ())),
                           precision=lax.Precision.HIGHEST,
                           preferred_element_type=jnp.float32)
    hist = hist + wb_ref[...]
    diff = pp - hist[:, None, :]
    d2 = jnp.sum(diff * diff, axis=-1)
    feats = -jnp.sqrt(d2)
    m = jnp.max(feats, axis=-1, keepdims=True)
    ex = jnp.exp(feats - m)
    out_ref[...] = ex / jnp.sum(ex, axis=-1, keepdims=True)


def kernel(persona, history, word_emb, pos_emb, tok_type_emb, ln_g, ln_b, W, b):
    ids = jnp.concatenate(
        [history[:, 1:].reshape(B, TOK),
         persona[:, :, 1:].reshape(B * P, TOK)], axis=0).astype(jnp.int32)
    ids = ids.reshape(NSEQ, NQ, QT).transpose(1, 0, 2).reshape(32, SPT, QT)
    c = (pos_emb[2:2 + TOK] + tok_type_emb[0]).reshape(NQ, QT, H)

    gathered = _make_sc_gather()(ids, word_emb)

    pooled = pl.pallas_call(
        _pool_body,
        grid=(NSEQ // BS,),
        in_specs=[
            pl.BlockSpec((NQ, BS, QT, H), lambda i: (0, i, 0, 0)),
            pl.BlockSpec((NQ, QT, H), lambda i: (0, 0, 0)),
            pl.BlockSpec((1, H), lambda i: (0, 0)),
            pl.BlockSpec((1, H), lambda i: (0, 0)),
        ],
        out_specs=pl.BlockSpec((BS, H), lambda i: (i, 0)),
        out_shape=jax.ShapeDtypeStruct((NSEQ, H), jnp.float32),
    )(gathered, c, ln_g.reshape(1, H), ln_b.reshape(1, H))

    return pl.pallas_call(
        _tail_body,
        out_shape=jax.ShapeDtypeStruct((B, P), jnp.float32),
    )(pooled, W, b)


# speedup vs baseline: 1.0583x; 1.0521x over previous
import jax
import jax.numpy as jnp
from jax import lax
from jax.experimental import pallas as pl
from jax.experimental.pallas import tpu as pltpu
from jax.experimental.pallas import tpu_sc as plsc

V = 50265
H = 768
B, P, T = 16, 8, 129
NSEQ = B + B * P
TOK = T - 1
NQ = 4
QT = TOK // NQ
SPT = NSEQ // 8
SPC = 2
NCH = SPT // SPC
CR = SPC * QT
BS = 24


def _sc_gather_body(ids_hbm, tab_hbm, out_hbm, ids_v, rows_v,
                    g0, g1, w0, w1):
    wid = lax.axis_index("c") * 16 + lax.axis_index("s")
    q = wid // 8
    row_base = (wid % 8) * (SPT * QT)

    pltpu.sync_copy(ids_hbm.at[wid], ids_v)
    gsem = (g0, g1)
    wsem = (w0, w1)

    def wdst(j):
        return out_hbm.at[q, pl.ds(row_base + j * CR, CR)]

    pltpu.async_copy(tab_hbm.at[ids_v.at[0]], rows_v.at[0], g0)
    pltpu.async_copy(tab_hbm.at[ids_v.at[1]], rows_v.at[1], g1)

    for j in range(NCH):
        buf = j % 2
        pltpu.make_async_copy(
            tab_hbm.at[ids_v.at[j]], rows_v.at[buf], gsem[buf]).wait()
        pltpu.async_copy(rows_v.at[buf], wdst(j), wsem[buf])
        if j + 2 < NCH:
            pltpu.make_async_copy(rows_v.at[buf], wdst(j), wsem[buf]).wait()
            pltpu.async_copy(
                tab_hbm.at[ids_v.at[j + 2]], rows_v.at[buf], gsem[buf])

    for j in (NCH - 2, NCH - 1):
        buf = j % 2
        pltpu.make_async_copy(rows_v.at[buf], wdst(j), wsem[buf]).wait()


def _make_sc_gather():
    mesh = plsc.VectorSubcoreMesh(core_axis_name="c", subcore_axis_name="s")
    return pl.kernel(
        _sc_gather_body,
        out_type=jax.ShapeDtypeStruct((NQ, NSEQ * QT, H), jnp.float32),
        mesh=mesh,
        scratch_types=[
            pltpu.VMEM((NCH, CR), jnp.int32),
            pltpu.VMEM((2, CR, H), jnp.float32),
            pltpu.SemaphoreType.DMA,
            pltpu.SemaphoreType.DMA,
            pltpu.SemaphoreType.DMA,
            pltpu.SemaphoreType.DMA,
        ],
    )


def _pool_body(g_ref, c_ref, gam_ref, bet_ref, out_ref):
    e = g_ref[...].reshape(NQ, BS, QT, H) + c_ref[...][:, None]
    mu = jnp.mean(e, axis=-1, keepdims=True)
    var = jnp.mean(e * e, axis=-1, keepdims=True) - mu * mu
    w = lax.rsqrt(var + jnp.float32(1e-5))
    s = jnp.sum(e * w, axis=(0, 2)) - jnp.sum(
        mu * w, axis=(0, 2))
    out_ref[...] = (s * jnp.float32(1.0 / TOK)) * gam_ref[...] + bet_ref[...]


def _tail_body(pooled_ref, w_ref, wb_ref, out_ref):
    pooled = pooled_ref[...]
    ph = pooled[:B]
    pp = pooled[B:].reshape(B, P, H)
    hist = lax.dot_general(ph, w_ref[...], (((1,), (1,)), ((), ())),
                           precision=lax.Precision.HIGHEST,
                           preferred_element_type=jnp.float32)
    hist = hist + wb_ref[...]
    diff = pp - hist[:, None, :]
    d2 = jnp.sum(diff * diff, axis=-1)
    feats = -jnp.sqrt(d2)
    m = jnp.max(feats, axis=-1, keepdims=True)
    ex = jnp.exp(feats - m)
    out_ref[...] = ex / jnp.sum(ex, axis=-1, keepdims=True)


def kernel(persona, history, word_emb, pos_emb, tok_type_emb, ln_g, ln_b, W, b):
    ids = jnp.concatenate(
        [history[:, 1:].reshape(B, TOK),
         persona[:, :, 1:].reshape(B * P, TOK)], axis=0).astype(jnp.int32)
    ids = ids.reshape(NSEQ, NQ, QT).transpose(1, 0, 2).reshape(32, NCH, CR)
    c = (pos_emb[2:2 + TOK] + tok_type_emb[0]).reshape(NQ, QT, H)

    gathered = _make_sc_gather()(ids, word_emb)

    pooled = pl.pallas_call(
        _pool_body,
        grid=(NSEQ // BS,),
        in_specs=[
            pl.BlockSpec((NQ, BS * QT, H), lambda i: (0, i, 0)),
            pl.BlockSpec((NQ, QT, H), lambda i: (0, 0, 0)),
            pl.BlockSpec((1, H), lambda i: (0, 0)),
            pl.BlockSpec((1, H), lambda i: (0, 0)),
        ],
        out_specs=pl.BlockSpec((BS, H), lambda i: (i, 0)),
        out_shape=jax.ShapeDtypeStruct((NSEQ, H), jnp.float32),
    )(gathered, c, ln_g.reshape(1, H), ln_b.reshape(1, H))

    return pl.pallas_call(
        _tail_body,
        out_shape=jax.ShapeDtypeStruct((B, P), jnp.float32),
    )(pooled, W, b)


# speedup vs baseline: 1.0810x; 1.0215x over previous
import jax
import jax.numpy as jnp
from jax import lax
from jax.experimental import pallas as pl
from jax.experimental.pallas import tpu as pltpu
from jax.experimental.pallas import tpu_sc as plsc

V = 50265
H = 768
B, P, T = 16, 8, 129
NSEQ = B + B * P
TOK = T - 1
NQ = 4
QT = TOK // NQ
SPT = NSEQ // 8
SPC = 2
NCH = SPT // SPC
CR = SPC * QT
BS = 24


def _sc_gather_body(ids_hbm, tab_hbm, out_hbm, ids_v, rows_v,
                    g0, g1, w0, w1):
    wid = lax.axis_index("c") * 16 + lax.axis_index("s")
    q = wid // 8
    row_base = (wid % 8) * (SPT * QT)

    pltpu.sync_copy(ids_hbm.at[wid], ids_v)
    gsem = (g0, g1)
    wsem = (w0, w1)

    def wdst(j):
        return out_hbm.at[q, pl.ds(row_base + j * CR, CR)]

    pltpu.async_copy(tab_hbm.at[ids_v.at[0]], rows_v.at[0], g0)
    pltpu.async_copy(tab_hbm.at[ids_v.at[1]], rows_v.at[1], g1)

    for j in range(NCH):
        buf = j % 2
        pltpu.make_async_copy(
            tab_hbm.at[ids_v.at[j]], rows_v.at[buf], gsem[buf]).wait()
        pltpu.async_copy(rows_v.at[buf], wdst(j), wsem[buf])
        if j + 2 < NCH:
            pltpu.make_async_copy(rows_v.at[buf], wdst(j), wsem[buf]).wait()
            pltpu.async_copy(
                tab_hbm.at[ids_v.at[j + 2]], rows_v.at[buf], gsem[buf])

    for j in (NCH - 2, NCH - 1):
        buf = j % 2
        pltpu.make_async_copy(rows_v.at[buf], wdst(j), wsem[buf]).wait()


def _make_sc_gather():
    mesh = plsc.VectorSubcoreMesh(core_axis_name="c", subcore_axis_name="s")
    return pl.kernel(
        _sc_gather_body,
        out_type=jax.ShapeDtypeStruct((NQ, NSEQ * QT, H), jnp.float32),
        mesh=mesh,
        scratch_types=[
            pltpu.VMEM((NCH, CR), jnp.int32),
            pltpu.VMEM((2, CR, H), jnp.float32),
            pltpu.SemaphoreType.DMA,
            pltpu.SemaphoreType.DMA,
            pltpu.SemaphoreType.DMA,
            pltpu.SemaphoreType.DMA,
        ],
    )


def _pool_tail_body(g_ref, c_ref, gam_ref, bet_ref, w_ref, wb_ref,
                    out_ref, acc_ref):
    i = pl.program_id(0)
    e = g_ref[...].reshape(NQ, BS, QT, H) + c_ref[...][:, None]
    mu = jnp.mean(e, axis=-1, keepdims=True)
    var = jnp.mean(e * e, axis=-1, keepdims=True) - mu * mu
    w = lax.rsqrt(var + jnp.float32(1e-5))
    s = jnp.sum(e * w, axis=(0, 2)) - jnp.sum(
        mu * w, axis=(0, 2))
    acc_ref[pl.ds(i * BS, BS)] = (
        s * jnp.float32(1.0 / TOK)) * gam_ref[...] + bet_ref[...]

    @pl.when(i == NSEQ // BS - 1)
    def _tail():
        pooled = acc_ref[...]
        ph = pooled[:B]
        pp = pooled[B:].reshape(B, P, H)
        hist = lax.dot_general(ph, w_ref[...], (((1,), (1,)), ((), ())),
                               precision=lax.Precision.HIGHEST,
                               preferred_element_type=jnp.float32)
        hist = hist + wb_ref[...]
        diff = pp - hist[:, None, :]
        d2 = jnp.sum(diff * diff, axis=-1)
        feats = -jnp.sqrt(d2)
        m = jnp.max(feats, axis=-1, keepdims=True)
        ex = jnp.exp(feats - m)
        out_ref[...] = ex / jnp.sum(ex, axis=-1, keepdims=True)


def kernel(persona, history, word_emb, pos_emb, tok_type_emb, ln_g, ln_b, W, b):
    ids = jnp.concatenate(
        [history[:, 1:].reshape(B, TOK),
         persona[:, :, 1:].reshape(B * P, TOK)], axis=0).astype(jnp.int32)
    ids = ids.reshape(NSEQ, NQ, QT).transpose(1, 0, 2).reshape(32, NCH, CR)
    c = (pos_emb[2:2 + TOK] + tok_type_emb[0]).reshape(NQ, QT, H)

    gathered = _make_sc_gather()(ids, word_emb)

    return pl.pallas_call(
        _pool_tail_body,
        grid=(NSEQ // BS,),
        in_specs=[
            pl.BlockSpec((NQ, BS * QT, H), lambda i: (0, i, 0)),
            pl.BlockSpec((NQ, QT, H), lambda i: (0, 0, 0)),
            pl.BlockSpec((1, H), lambda i: (0, 0)),
            pl.BlockSpec((1, H), lambda i: (0, 0)),
            pl.BlockSpec((H, H), lambda i: (0, 0)),
            pl.BlockSpec((1, H), lambda i: (0, 0)),
        ],
        out_specs=pl.BlockSpec((B, P), lambda i: (0, 0)),
        out_shape=jax.ShapeDtypeStruct((B, P), jnp.float32),
        scratch_shapes=[pltpu.VMEM((NSEQ, H), jnp.float32)],
    )(gathered, c, ln_g.reshape(1, H), ln_b.reshape(1, H),
      W, b.reshape(1, H))
